# Initial kernel scaffold; baseline (speedup 1.0000x reference)
import jax
import jax.numpy as jnp
from jax.experimental import pallas as pl


def kernel(x, patch_w, patch_b, fc1_w, fc1_b, gconv_w, gconv_b, fc2_w, fc2_b, ffn1_w, ffn1_b, ffn2_w, ffn2_b, bn_g, bn_b):
    raise NotImplementedError("write your pallas kernel here")


# speedup vs baseline: 6.1611x; 6.1611x over previous
import functools

import jax
import jax.numpy as jnp
import numpy as np
from jax.experimental import pallas as pl
from jax.experimental.pallas import tpu as pltpu

_BIG = 3.0e38
_MASK16 = -65536


def _patch_kernel(a_ref, w_ref, b_ref, o_ref):
    a = a_ref[:]
    w = w_ref[:]
    kd = a.shape[1]
    acc = jnp.dot(a[:, :256], w[:256], preferred_element_type=jnp.float32)
    for s in range(256, kd, 256):
        acc = acc + jnp.dot(a[:, s:s + 256], w[s:s + 256],
                            preferred_element_type=jnp.float32)
    o_ref[:] = acc + b_ref[:]


def _split3(v):
    vi = jax.lax.bitcast_convert_type(v, jnp.int32)
    p1 = jax.lax.bitcast_convert_type(vi & _MASK16, jnp.float32)
    r1 = v - p1
    r1i = jax.lax.bitcast_convert_type(r1, jnp.int32)
    p2 = jax.lax.bitcast_convert_type(r1i & _MASK16, jnp.float32)
    p3 = r1 - p2
    return (p1.astype(jnp.bfloat16), p2.astype(jnp.bfloat16),
            p3.astype(jnp.bfloat16))


def _layer_kernel(h_ref, fc1w, fc1b, gcw, gcb, fc2w, fc2b, f1w, f1b, f2w,
                  f2b, bng, bnb, out_ref, *, K, bn):
    Bn, N, C = h_ref.shape
    h = h_ref[:]
    hf = h.reshape(Bn * N, C)

    y = jnp.dot(hf, fc1w[:], preferred_element_type=jnp.float32) + fc1b[:]

    jiota = jax.lax.broadcasted_iota(jnp.int32, (N, N), 1)

    hmid = []
    for b in range(Bn):
        yb = y[b * N:(b + 1) * N, :]
        yt = jnp.transpose(yb)
        x2row = jnp.sum(yt * yt, axis=0, keepdims=True)
        x2col = jnp.transpose(x2row)
        gram = jax.lax.dot_general(
            yb, yb, (((1,), (1,)), ((), ())),
            preferred_element_type=jnp.float32)
        d = x2col + x2row - 2.0 * gram

        p1b, p2b, p3b = _split3(yb)

        def topk_body(_, carry):
            d, M = carry
            rmin = jnp.min(d, axis=1, keepdims=True)
            eq = d == rmin
            jarg = jnp.min(
                jnp.where(eq, jiota, jnp.int32(2**30)),
                axis=1, keepdims=True)
            onehot = jiota == jarg
            oh = onehot.astype(jnp.bfloat16)
            g = (jnp.dot(oh, p1b, preferred_element_type=jnp.float32)
                 + jnp.dot(oh, p2b, preferred_element_type=jnp.float32)) \
                + jnp.dot(oh, p3b, preferred_element_type=jnp.float32)
            M = jnp.maximum(M, g)
            d = jnp.where(onehot, jnp.float32(_BIG), d)
            return d, M

        M0 = jnp.full((N, C), -_BIG, dtype=jnp.float32)
        _, M = jax.lax.fori_loop(0, K, topk_body, (d, M0))

        cat = jnp.concatenate([yb, M - yb], axis=1)
        gc = jax.nn.gelu(
            jnp.dot(cat, gcw[:], preferred_element_type=jnp.float32)
            + gcb[:])
        y2 = jnp.dot(gc, fc2w[:], preferred_element_type=jnp.float32) + fc2b[:]
        hmid.append(h[b] + y2)

    hm = jnp.concatenate([v[None] for v in hmid], axis=0).reshape(Bn * N, C)
    t = jax.nn.gelu(
        jnp.dot(hm, f1w[:], preferred_element_type=jnp.float32) + f1b[:])
    hn = hm + jnp.dot(t, f2w[:], preferred_element_type=jnp.float32) + f2b[:]

    if not bn:
        out_ref[:] = hn.reshape(Bn, N, C)
        return

    gl = [jax.nn.gelu(jnp.transpose(hn[b * N:(b + 1) * N, :]))
          for b in range(Bn)]
    cw = min(128, N)
    scale = jnp.float32(1.0 / (Bn * N))

    def stat_sum(tiles):
        acc = tiles[0]
        for c in tiles[1:]:
            acc = acc + c
        tr = jnp.transpose(acc)
        return jnp.transpose(jnp.sum(tr, axis=0, keepdims=True))

    chunks_bt = [gl[b][:, s:s + cw] for b in range(Bn) for s in range(0, N, cw)]
    mu = stat_sum(chunks_bt) * scale
    sqfull = [(g - mu) * (g - mu) for g in gl]
    sq_tb = [sqfull[b][:, s:s + cw] for s in range(0, N, cw) for b in range(Bn)]
    var = stat_sum(sq_tb) * scale
    den = jnp.sqrt(var + jnp.float32(1e-5))
    gcol = jnp.transpose(bng[:])
    bcol = jnp.transpose(bnb[:])
    for b in range(Bn):
        out_ref[b, :, :] = jnp.transpose(((gl[b] - mu) / den) * gcol + bcol)


def _run(x, patch_w, patch_b, fc1_w, fc1_b, gconv_w, gconv_b, fc2_w, fc2_b,
         ffn1_w, ffn1_b, ffn2_w, ffn2_b, bn_g, bn_b, *, K):
    Bn, Ci, H, _ = x.shape
    C = patch_w.shape[0]
    P = int(np.sqrt(patch_w.shape[1] // Ci))
    hw = H // P
    N = hw * hw
    L = fc1_w.shape[0]

    xp = (x.reshape(Bn, Ci, hw, P, hw, P)
           .transpose(0, 2, 4, 1, 3, 5)
           .reshape(Bn * N, Ci * P * P))

    h = pl.pallas_call(
        _patch_kernel,
        out_shape=jax.ShapeDtypeStruct((Bn * N, C), jnp.float32),
    )(xp, patch_w.T, patch_b.reshape(1, C))
    h = h.reshape(Bn, N, C)

    layer_bn = pl.pallas_call(
        functools.partial(_layer_kernel, K=K, bn=True),
        out_shape=jax.ShapeDtypeStruct((Bn, N, C), jnp.float32),
    )
    layer_last = pl.pallas_call(
        functools.partial(_layer_kernel, K=K, bn=False),
        out_shape=jax.ShapeDtypeStruct((Bn, N, C), jnp.float32),
    )

    for i in range(L):
        layer = layer_bn if i < L - 1 else layer_last
        gi = min(i, L - 2)
        h = layer(h,
                  fc1_w[i].T, fc1_b[i][None, :],
                  gconv_w[i].T, gconv_b[i][None, :],
                  fc2_w[i].T, fc2_b[i][None, :],
                  ffn1_w[i].T, ffn1_b[i][None, :],
                  ffn2_w[i].T, ffn2_b[i][None, :],
                  bn_g[gi][None, :], bn_b[gi][None, :])

    return h.transpose(0, 2, 1).reshape(Bn, C, hw, hw)


def kernel(x, patch_w, patch_b, fc1_w, fc1_b, gconv_w, gconv_b, fc2_w, fc2_b,
           ffn1_w, ffn1_b, ffn2_w, ffn2_b, bn_g, bn_b):
    return _run(x, patch_w, patch_b, fc1_w, fc1_b, gconv_w, gconv_b,
                fc2_w, fc2_b, ffn1_w, ffn1_b, ffn2_w, ffn2_b, bn_g, bn_b,
                K=16)
